# baseline (speedup 1.0000x reference)
import functools

import jax
import jax.numpy as jnp
from jax import lax
from jax.experimental import pallas as pl
from jax.experimental.pallas import tpu as pltpu
from jax.experimental.pallas import tpu_sc as plsc

SEQ = 512
CLS_ID = 101
SEP_ID = 102
EMB_D = 128
ZROW = 30522
VOCAB_PAD = 30528
NC = 2
NS = 16
NW = NC * NS
K = 128


def _pack_body(body_ref, len_ref, packed_ref, mask_ref, idx_ref):
    bm = body_ref.shape[0]
    pos = lax.broadcasted_iota(jnp.int32, (bm, SEQ), 1)
    L = len_ref[...]
    body = body_ref[...]
    packed = jnp.where(pos == 0, CLS_ID,
             jnp.where(pos <= L, body,
             jnp.where(pos == L + 1, SEP_ID, 0)))
    mask = pos <= L + 1
    packed_ref[...] = packed
    mask_ref[...] = mask.astype(jnp.int32)
    idx_ref[...] = jnp.where(mask, packed, ZROW)


def _pack_call(body, lengths2d):
    B = body.shape[0]
    bm = 256
    grid = B // bm
    return pl.pallas_call(
        _pack_body,
        grid=(grid,),
        in_specs=[pl.BlockSpec((bm, SEQ), lambda i: (i, 0)),
                  pl.BlockSpec((bm, 1), lambda i: (i, 0))],
        out_specs=[pl.BlockSpec((bm, SEQ), lambda i: (i, 0))] * 3,
        out_shape=[jax.ShapeDtypeStruct((B, SEQ), jnp.int32)] * 3,
    )(body, lengths2d)


NBUF = 4


def _sc_gather(idx_flat, table_pad):
    BT = idx_flat.shape[0]
    span = BT // NW
    C = span // K
    R = C // NBUF
    mesh = plsc.VectorSubcoreMesh(core_axis_name="c", subcore_axis_name="s")

    @functools.partial(
        pl.kernel, mesh=mesh,
        out_type=jax.ShapeDtypeStruct((BT, EMB_D), jnp.float32),
        scratch_types=(
            [pltpu.VMEM((span,), jnp.int32)]
            + [pltpu.VMEM((K, EMB_D), jnp.float32) for _ in range(NBUF)]
            + [pltpu.SemaphoreType.DMA for _ in range(2 * NBUF)]
        ),
    )
    def k(idx_hbm, table_hbm, out_hbm, idx_v, *rest):
        bufs = rest[:NBUF]
        gsem = rest[NBUF:2 * NBUF]
        ssem = rest[2 * NBUF:3 * NBUF]
        wid = lax.axis_index("s") * NC + lax.axis_index("c")
        base = wid * span
        pltpu.sync_copy(idx_hbm.at[pl.ds(base, span)], idx_v)

        def g_start(c, j):
            pltpu.async_copy(table_hbm.at[idx_v.at[pl.ds(c * K, K)]],
                             bufs[j], gsem[j])

        def g_wait(j):
            pltpu.make_async_copy(table_hbm.at[idx_v.at[pl.ds(0, K)]],
                                  bufs[j], gsem[j]).wait()

        def s_start(c, j):
            pltpu.async_copy(bufs[j], out_hbm.at[pl.ds(base + c * K, K)],
                             ssem[j])

        def s_wait(j):
            pltpu.make_async_copy(bufs[j], out_hbm.at[pl.ds(0, K)],
                                  ssem[j]).wait()

        for j in range(NBUF):
            g_start(j, j)

        def outer(i, carry):
            cb = i * NBUF
            for j in range(NBUF):
                g_wait(j)
                s_start(cb + j, j)

            @pl.when(i + 1 < R)
            def _():
                for j in range(NBUF):
                    s_wait(j)
                    g_start(cb + NBUF + j, j)

            return carry

        lax.fori_loop(0, R, outer, 0)
        for j in range(NBUF):
            s_wait(j)

    return k(idx_flat, table_pad)


def kernel(token_ids, lengths, table):
    B = token_ids.shape[0]
    body = jnp.pad(token_ids, ((0, 0), (1, 1)))
    packed, maski, idx = _pack_call(body, lengths[:, None])
    table_pad = jnp.pad(table, ((0, VOCAB_PAD - table.shape[0]), (0, 0)))
    emb = _sc_gather(idx.reshape(-1), table_pad).reshape(B, SEQ, EMB_D)
    segment_ids = jnp.zeros((B, SEQ), jnp.int32)
    return packed, segment_ids, maski.astype(jnp.bool_), emb


# speedup vs baseline: 30.1020x; 30.1020x over previous
import functools

import jax
import jax.numpy as jnp
from jax import lax
from jax.experimental import pallas as pl
from jax.experimental.pallas import tpu as pltpu
from jax.experimental.pallas import tpu_sc as plsc

SEQ = 512
CLS_ID = 101
SEP_ID = 102
EMB_D = 128
ZBASE = 30522
VOCAB_PAD = 31040
NC = 2
NS = 16
NW = NC * NS
K = 128


def _pack_body(body_ref, len_ref, packed_ref, mask_ref, idx_ref):
    bm = body_ref.shape[0]
    pos = lax.broadcasted_iota(jnp.int32, (bm, SEQ), 1)
    L = len_ref[...]
    body = body_ref[...]
    packed = jnp.where(pos == 0, CLS_ID,
             jnp.where(pos <= L, body,
             jnp.where(pos == L + 1, SEP_ID, 0)))
    mask = pos <= L + 1
    packed_ref[...] = packed
    mask_ref[...] = mask.astype(jnp.int32)
    idx_ref[...] = jnp.where(mask, packed, ZBASE + pos)


def _pack_call(body, lengths2d):
    B = body.shape[0]
    bm = 256
    grid = B // bm
    return pl.pallas_call(
        _pack_body,
        grid=(grid,),
        in_specs=[pl.BlockSpec((bm, SEQ), lambda i: (i, 0)),
                  pl.BlockSpec((bm, 1), lambda i: (i, 0))],
        out_specs=[pl.BlockSpec((bm, SEQ), lambda i: (i, 0))] * 3,
        out_shape=[jax.ShapeDtypeStruct((B, SEQ), jnp.int32)] * 3,
    )(body, lengths2d)


NBUF = 4


def _sc_gather(idx_flat, table_pad):
    BT = idx_flat.shape[0]
    span = BT // NW
    C = span // K
    R = C // NBUF
    mesh = plsc.VectorSubcoreMesh(core_axis_name="c", subcore_axis_name="s")

    @functools.partial(
        pl.kernel, mesh=mesh,
        out_type=jax.ShapeDtypeStruct((BT, EMB_D), jnp.float32),
        scratch_types=(
            [pltpu.VMEM((span,), jnp.int32)]
            + [pltpu.VMEM((K, EMB_D), jnp.float32) for _ in range(NBUF)]
            + [pltpu.SemaphoreType.DMA for _ in range(2 * NBUF)]
        ),
    )
    def k(idx_hbm, table_hbm, out_hbm, idx_v, *rest):
        bufs = rest[:NBUF]
        gsem = rest[NBUF:2 * NBUF]
        ssem = rest[2 * NBUF:3 * NBUF]
        wid = lax.axis_index("s") * NC + lax.axis_index("c")
        base = wid * span
        pltpu.sync_copy(idx_hbm.at[pl.ds(base, span)], idx_v)

        def g_start(c, j):
            pltpu.async_copy(table_hbm.at[idx_v.at[pl.ds(c * K, K)]],
                             bufs[j], gsem[j])

        def g_wait(j):
            pltpu.make_async_copy(table_hbm.at[idx_v.at[pl.ds(0, K)]],
                                  bufs[j], gsem[j]).wait()

        def s_start(c, j):
            pltpu.async_copy(bufs[j], out_hbm.at[pl.ds(base + c * K, K)],
                             ssem[j])

        def s_wait(j):
            pltpu.make_async_copy(bufs[j], out_hbm.at[pl.ds(0, K)],
                                  ssem[j]).wait()

        for j in range(NBUF):
            g_start(j, j)

        def outer(i, carry):
            cb = i * NBUF
            for j in range(NBUF):
                g_wait(j)
                s_start(cb + j, j)

            @pl.when(i + 1 < R)
            def _():
                for j in range(NBUF):
                    s_wait(j)
                    g_start(cb + NBUF + j, j)

            return carry

        lax.fori_loop(0, R, outer, 0)
        for j in range(NBUF):
            s_wait(j)

    return k(idx_flat, table_pad)


def kernel(token_ids, lengths, table):
    B = token_ids.shape[0]
    body = jnp.pad(token_ids, ((0, 0), (1, 1)))
    packed, maski, idx = _pack_call(body, lengths[:, None])
    table_pad = jnp.pad(table, ((0, VOCAB_PAD - table.shape[0]), (0, 0)))
    emb = _sc_gather(idx.reshape(-1), table_pad).reshape(B, SEQ, EMB_D)
    segment_ids = jnp.zeros((B, SEQ), jnp.int32)
    return packed, segment_ids, maski.astype(jnp.bool_), emb
